# Initial kernel scaffold; baseline (speedup 1.0000x reference)
import jax
import jax.numpy as jnp
from jax.experimental import pallas as pl


def kernel(x1, x2, cnn, emb, pos_embed, fm1, fm2, fusion):
    raise NotImplementedError("write your pallas kernel here")


# speedup vs baseline: 1.7456x; 1.7456x over previous
import numpy as np
import jax
import jax.numpy as jnp
from jax.experimental import pallas as pl
from jax.experimental.pallas import tpu as pltpu

_EPS = 1e-5
_B = 512
_BB = 64
_L = 49
_D = 64
_N = 16
_HW = 169
_CN = _BB * _D


def _pool_np(n_in, n_out):
    P = np.zeros((n_out, n_in), np.float32)
    for i in range(n_out):
        s = (i * n_in) // n_out
        e = -((-(i + 1) * n_in) // n_out)
        P[i, s:e] = 1.0 / (e - s)
    return P


_KP = np.kron(_pool_np(13, 7), _pool_np(13, 7))


def _conv_body(x_ref, w1_ref, b1_ref, w2_ref, b2_ref, outa_ref, outb_ref):
    x = x_ref[...]
    xp = jnp.pad(x, ((1, 1), (1, 1), (0, 0), (0, 0)))
    acc = None
    for dy in range(3):
        slab = jnp.concatenate(
            [xp[dy:dy + 13, dx:dx + 13].reshape(_HW * _BB, 31)
             for dx in range(3)], axis=-1)
        t = jnp.dot(slab, w1_ref[...][dy * 93:(dy + 1) * 93, :],
                    preferred_element_type=jnp.float32)
        acc = t if acc is None else acc + t
    h1 = jax.nn.relu(acc + b1_ref[...]).astype(jnp.bfloat16)
    hp = jnp.pad(h1.reshape(13, 13, _BB, 64), ((1, 1), (1, 1), (0, 0), (0, 0)))
    acc2 = None
    for dy in range(3):
        slab = jnp.concatenate(
            [hp[dy:dy + 13, dx:dx + 13].reshape(_HW * _BB, 64)
             for dx in range(3)], axis=-1)
        t = jnp.dot(slab, w2_ref[...][dy * 192:(dy + 1) * 192, :],
                    preferred_element_type=jnp.float32)
        acc2 = t if acc2 is None else acc2 + t
    h2 = jax.nn.relu(acc2 + b2_ref[...]).astype(jnp.bfloat16)
    outa_ref[...] = h2[:, :64].reshape(_HW, _BB, 64)
    outb_ref[...] = h2[:, 64:].reshape(_HW, _BB, 64)


def _pool_body(xa_ref, xb_ref, ma_ref, mb_ref, pa_ref, pb_ref):
    pa_ref[...] = jnp.dot(ma_ref[...], xa_ref[...],
                          preferred_element_type=jnp.float32)
    pb_ref[...] = jnp.dot(mb_ref[...], xb_ref[...],
                          preferred_element_type=jnp.float32)


def _lnk(x2, g, b):
    J = jnp.full((_D, _D), 1.0 / _D, jnp.float32)
    m = jnp.dot(x2, J, preferred_element_type=jnp.float32)
    m2 = jnp.dot(x2 * x2, J, preferred_element_type=jnp.float32)
    r = jax.lax.rsqrt(m2 - m * m + _EPS)
    return (x2 - m) * r * g + b


def _mamba(x3, p, scr):
    (ln_g, ln_b, w_in, conv_w, conv_b, wbig, b_dt, at, dskip, w_out,
     b_out) = p
    dt_s, u_s, ys_s, b_s, c_s, pad_s = scr
    L, bb, d = x3.shape
    h = _lnk(x3.reshape(L * bb, d), ln_g[...], ln_b[...])
    xz = jnp.dot(h, w_in[...], preferred_element_type=jnp.float32)
    z = xz[:, d:]
    xc3 = xz[:, :d].reshape(L, bb, d)
    cw = conv_w[...]
    pad_s[0:3] = jnp.zeros((3, bb, d), jnp.float32)
    pad_s[3:] = xc3
    y3 = xc3 * cw[3][None, None, :]
    for k in range(3):
        y3 = y3 + pad_s[k:k + L] * cw[k][None, None, :]
    y3 = y3 + conv_b[...].reshape(1, 1, d)
    y3 = y3 * jax.nn.sigmoid(y3)
    y2 = y3.reshape(L * bb, d)
    big = jnp.dot(y2, wbig[...], preferred_element_type=jnp.float32)
    dt2 = jax.nn.softplus(big[:, :d] + b_dt[...])
    dt_s[...] = dt2.reshape(L, bb, d)
    u_s[...] = (dt2 * y2).reshape(L, bb, d)
    b_s[...] = big[:, d:d + _N].reshape(L, bb, _N)
    c_s[...] = big[:, d + _N:d + 2 * _N].reshape(L, bb, _N)
    atv = at[...]

    def step(t, S):
        dtt = dt_s[pl.ds(t, 1)].reshape(bb, d)
        ut = u_s[pl.ds(t, 1)].reshape(bb, d)
        bt = b_s[pl.ds(t, 1)].reshape(bb, _N)
        ct = c_s[pl.ds(t, 1)].reshape(bb, _N)
        dA = jnp.exp(dtt[:, None, :] * atv[None, :, :])
        S = S * dA + ut[:, None, :] * bt[:, :, None]
        yt = jnp.sum(S * ct[:, :, None], axis=1)
        ys_s[pl.ds(t, 1)] = yt.reshape(1, bb, d)
        return S

    jax.lax.fori_loop(0, L, step, jnp.zeros((bb, _N, d), jnp.float32))
    ys3 = ys_s[...] + y3 * dskip[...].reshape(1, 1, d)
    gated = ys3.reshape(L * bb, d) * (z * jax.nn.sigmoid(z))
    o2 = jnp.dot(gated, w_out[...], preferred_element_type=jnp.float32) + b_out[...]
    return x3 + o2.reshape(L, bb, d)


def _fuse_body(pa_ref, pb_ref, *rest):
    (bias2a, bias2b, lin1a_w, lin1a_b, lin1b_w, lin1b_b,
     norm_a_g, norm_a_b, norm_b_g, norm_b_b, fus_w, fus_b) = rest[:12]
    pma = rest[12:23]
    pspec = rest[23:34]
    pmb = rest[34:45]
    out_ref = rest[45]
    scr = rest[46:52]

    def fuse(x3, lw, lb, ng, nb, mparams):
        L, bb, d = x3.shape
        xp2 = jnp.dot(x3.reshape(L * bb, d), lw[...],
                      preferred_element_type=jnp.float32) + lb[...]
        m3 = xp2.reshape(L, bb, d)
        for mp in mparams:
            m3 = _mamba(m3, mp, scr)
        x3 = x3 + m3
        h = jax.nn.relu(_lnk(x3.reshape(L * bb, d), ng[...], nb[...]))
        return h

    xa3 = pa_ref[...] + bias2a[...][:, None, :]
    xb3 = pb_ref[...] + bias2b[...][:, None, :]
    fa = fuse(xa3, lin1a_w, lin1a_b, norm_a_g, norm_a_b, (pma, pspec))
    fb = fuse(xb3, lin1b_w, lin1b_b, norm_b_g, norm_b_b, (pmb,))
    cat = jnp.concatenate([fa, fb], axis=-1)
    o = jnp.dot(cat, fus_w[...], preferred_element_type=jnp.float32) + fus_b[...]
    out_ref[...] = jnp.swapaxes(o.reshape(_L, _BB, _D), 0, 1)


def _wspec(a):
    nd = a.ndim
    return pl.BlockSpec(a.shape, lambda i, _nd=nd: (0,) * _nd)


def _fold_bn(w, b, g, beta, m, v):
    s = g * jax.lax.rsqrt(v + _EPS)
    return w * s[:, None, None, None], (b - m) * s + beta


def _prep_mamba(p):
    dt_rank = p['w_dt'].shape[0]
    wbig = jnp.concatenate(
        [p['w_x'][:, :dt_rank] @ p['w_dt'],
         p['w_x'][:, dt_rank:dt_rank + _N],
         p['w_x'][:, dt_rank + _N:dt_rank + 2 * _N]], axis=1)
    at = -jnp.exp(p['a_log']).T
    r = lambda a: a.reshape(1, -1)
    return [r(p['ln_g']), r(p['ln_b']), p['w_in'], p['conv_w'].T,
            r(p['conv_b']), wbig, r(p['b_dt']), at, r(p['dskip']),
            p['w_out'], r(p['b_out'])]


def kernel(x1, x2, cnn, emb, pos_embed, fm1, fm2, fusion):
    bf16 = jnp.bfloat16
    w1a, b1a = _fold_bn(cnn['c1_w'], cnn['c1_b'], cnn['c1_g'],
                        cnn['c1_beta'], cnn['c1_m'], cnn['c1_v'])
    w1b, b1b = _fold_bn(cnn['c2_w'], cnn['c2_b'], cnn['c2_g'],
                        cnn['c2_beta'], cnn['c2_m'], cnn['c2_v'])
    w2a, b2a = _fold_bn(cnn['c1b_w'], cnn['c1b_b'], cnn['c1b_g'],
                        cnn['c1b_beta'], cnn['c1b_m'], cnn['c1b_v'])
    w2b, b2b = _fold_bn(cnn['c2b_w'], cnn['c2b_b'], cnn['c2b_g'],
                        cnn['c2b_beta'], cnn['c2b_m'], cnn['c2b_v'])
    w1c = jnp.zeros((3, 3, 31, 64), jnp.float32)
    w1c = w1c.at[:, :, :30, :32].set(w1a.transpose(2, 3, 1, 0))
    w1c = w1c.at[:, :, 30:, 32:].set(w1b.transpose(2, 3, 1, 0))
    w1c = w1c.reshape(9 * 31, 64).astype(bf16)
    b1c = jnp.concatenate([b1a, b1b]).reshape(1, 64)
    w2c = jnp.zeros((3, 3, 64, 128), jnp.float32)
    w2c = w2c.at[:, :, :32, :64].set(w2a.transpose(2, 3, 1, 0))
    w2c = w2c.at[:, :, 32:, 64:].set(w2b.transpose(2, 3, 1, 0))
    w2c = w2c.reshape(576, 128).astype(bf16)
    b2c = jnp.concatenate([b2a, b2b]).reshape(1, 128)

    xcat = jnp.concatenate(
        [jnp.transpose(x1, (2, 3, 0, 1)),
         jnp.transpose(x2, (2, 3, 0, 1))], axis=-1).astype(bf16)

    kp = jnp.asarray(_KP)
    m1ta = (emb['w1'].T @ kp).astype(bf16)
    m1tb = (emb['w2'].T @ kp).astype(bf16)
    bias2a = emb['b1'][:, None] + pos_embed[0, :_L, :]
    bias2b = emb['b2'][:, None] + pos_embed[0, :_L, :]

    pma = _prep_mamba(fm1['ms'])
    pspec = _prep_mamba(fm1['spec'])
    pmb = _prep_mamba(fm2['ms'])
    r = lambda a: a.reshape(1, -1)
    weights = ([bias2a, bias2b, fm1['lin1_w'], r(fm1['lin1_b']),
                fm2['lin1_w'], r(fm2['lin1_b']),
                r(fm1['norm_g']), r(fm1['norm_b']),
                r(fm2['norm_g']), r(fm2['norm_b']),
                fusion['w'], r(fusion['b'])] + pma + pspec + pmb)

    nblk = _B // _BB
    cparams = pltpu.CompilerParams(dimension_semantics=("parallel",),
                                   vmem_limit_bytes=56 * 2 ** 20)

    a2, b2_ = pl.pallas_call(
        _conv_body,
        grid=(nblk,),
        in_specs=[pl.BlockSpec((13, 13, _BB, 31), lambda i: (0, 0, i, 0)),
                  _wspec(w1c), _wspec(b1c), _wspec(w2c), _wspec(b2c)],
        out_specs=[pl.BlockSpec((_HW, _BB, 64), lambda i: (0, i, 0))] * 2,
        out_shape=[jax.ShapeDtypeStruct((_HW, _B, 64), bf16)] * 2,
        compiler_params=cparams,
    )(xcat, w1c, b1c, w2c, b2c)

    xa = a2.reshape(_HW, _B * _D)
    xb = b2_.reshape(_HW, _B * _D)
    pa, pb = pl.pallas_call(
        _pool_body,
        grid=(_B * _D // _CN,),
        in_specs=[pl.BlockSpec((_HW, _CN), lambda i: (0, i)),
                  pl.BlockSpec((_HW, _CN), lambda i: (0, i)),
                  _wspec(m1ta), _wspec(m1tb)],
        out_specs=[pl.BlockSpec((_L, _CN), lambda i: (0, i))] * 2,
        out_shape=[jax.ShapeDtypeStruct((_L, _B * _D), jnp.float32)] * 2,
        compiler_params=cparams,
    )(xa, xb, m1ta, m1tb)

    pa3 = pa.reshape(_L, _B, _D)
    pb3 = pb.reshape(_L, _B, _D)
    out = pl.pallas_call(
        _fuse_body,
        grid=(nblk,),
        in_specs=([pl.BlockSpec((_L, _BB, _D), lambda i: (0, i, 0))] * 2 +
                  [_wspec(w) for w in weights]),
        out_specs=pl.BlockSpec((_BB, _L, _D), lambda i: (i, 0, 0)),
        out_shape=jax.ShapeDtypeStruct((_B, _L, _D), jnp.float32),
        scratch_shapes=[pltpu.VMEM((_L, _BB, _D), jnp.float32)] * 3 +
                       [pltpu.VMEM((_L, _BB, _N), jnp.float32)] * 2 +
                       [pltpu.VMEM((_L + 3, _BB, _D), jnp.float32)],
        compiler_params=cparams,
    )(pa3, pb3, *weights)

    return out


# speedup vs baseline: 2.0007x; 1.1461x over previous
import numpy as np
import jax
import jax.numpy as jnp
from jax.experimental import pallas as pl
from jax.experimental.pallas import tpu as pltpu

_EPS = 1e-5
_B = 512
_BB = 32
_L = 49
_D = 64
_N = 16
_HW = 169
_CN = _BB * _D


def _pool_np(n_in, n_out):
    P = np.zeros((n_out, n_in), np.float32)
    for i in range(n_out):
        s = (i * n_in) // n_out
        e = -((-(i + 1) * n_in) // n_out)
        P[i, s:e] = 1.0 / (e - s)
    return P


_KP = np.kron(_pool_np(13, 7), _pool_np(13, 7))


def _conv_body(x_ref, w1_ref, b1_ref, w2_ref, b2_ref, outa_ref, outb_ref):
    x = x_ref[...]
    xp = jnp.pad(x, ((1, 1), (1, 1), (0, 0), (0, 0)))
    acc = None
    for dy in range(3):
        slab = jnp.concatenate(
            [xp[dy:dy + 13, dx:dx + 13].reshape(_HW * _BB, 31)
             for dx in range(3)], axis=-1)
        t = jnp.dot(slab, w1_ref[...][dy * 93:(dy + 1) * 93, :],
                    preferred_element_type=jnp.float32)
        acc = t if acc is None else acc + t
    h1 = jax.nn.relu(acc + b1_ref[...]).astype(jnp.bfloat16)
    hp = jnp.pad(h1.reshape(13, 13, _BB, 64), ((1, 1), (1, 1), (0, 0), (0, 0)))
    acc2 = None
    for dy in range(3):
        slab = jnp.concatenate(
            [hp[dy:dy + 13, dx:dx + 13].reshape(_HW * _BB, 64)
             for dx in range(3)], axis=-1)
        t = jnp.dot(slab, w2_ref[...][dy * 192:(dy + 1) * 192, :],
                    preferred_element_type=jnp.float32)
        acc2 = t if acc2 is None else acc2 + t
    h2 = jax.nn.relu(acc2 + b2_ref[...]).astype(jnp.bfloat16)
    outa_ref[...] = h2[:, :64].reshape(_HW, _BB, 64)
    outb_ref[...] = h2[:, 64:].reshape(_HW, _BB, 64)


def _pool_body(xa_ref, xb_ref, ma_ref, mb_ref, pa_ref, pb_ref):
    pa_ref[...] = jnp.dot(ma_ref[...], xa_ref[...],
                          preferred_element_type=jnp.float32)
    pb_ref[...] = jnp.dot(mb_ref[...], xb_ref[...],
                          preferred_element_type=jnp.float32)


def _lnk(x2, g, b):
    J = jnp.full((_D, _D), 1.0 / _D, jnp.float32)
    m = jnp.dot(x2, J, preferred_element_type=jnp.float32)
    m2 = jnp.dot(x2 * x2, J, preferred_element_type=jnp.float32)
    r = jax.lax.rsqrt(m2 - m * m + _EPS)
    return (x2 - m) * r * g + b


def _mamba(x3, p, scr):
    (ln_g, ln_b, w_in, conv_w, conv_b, wbig, b_dt, at, dskip, w_out,
     b_out) = p
    da_s, ub_s, ys_s, c_s, pad_s = scr
    L, bb, d = x3.shape
    h = _lnk(x3.reshape(L * bb, d), ln_g[...], ln_b[...])
    xz = jnp.dot(h, w_in[...], preferred_element_type=jnp.float32)
    z = xz[:, d:]
    xc3 = xz[:, :d].reshape(L, bb, d)
    cw = conv_w[...]
    pad_s[0:3] = jnp.zeros((3, bb, d), jnp.float32)
    pad_s[3:] = xc3
    y3 = xc3 * cw[3][None, None, :]
    for k in range(3):
        y3 = y3 + pad_s[k:k + L] * cw[k][None, None, :]
    y3 = y3 + conv_b[...].reshape(1, 1, d)
    y3 = y3 * jax.nn.sigmoid(y3)
    y2 = y3.reshape(L * bb, d)
    big = jnp.dot(y2, wbig[...], preferred_element_type=jnp.float32)
    dt2 = jax.nn.softplus(big[:, :d] + b_dt[...])
    dt3 = dt2.reshape(L, bb, d)
    u3 = (dt2 * y2).reshape(L, bb, d)
    bm3 = big[:, d:d + _N].reshape(L, bb, _N)
    atv = at[...]
    da_s[...] = jnp.exp(dt3[:, :, None, :] * atv[None, None, :, :])
    ub_s[...] = u3[:, :, None, :] * bm3[:, :, :, None]
    c_s[...] = big[:, d + _N:d + 2 * _N].reshape(L, bb, _N)

    def step7(i7, S):
        for k in range(7):
            t = i7 * 7 + k
            S = (S * da_s[pl.ds(t, 1)].reshape(bb, _N, d)
                 + ub_s[pl.ds(t, 1)].reshape(bb, _N, d))
            ct = c_s[pl.ds(t, 1)].reshape(bb, _N)
            ys_s[pl.ds(t, 1)] = jnp.sum(
                S * ct[:, :, None], axis=1).reshape(1, bb, d)
        return S

    jax.lax.fori_loop(0, 7, step7, jnp.zeros((bb, _N, d), jnp.float32))
    ys3 = ys_s[...] + y3 * dskip[...].reshape(1, 1, d)
    gated = ys3.reshape(L * bb, d) * (z * jax.nn.sigmoid(z))
    o2 = jnp.dot(gated, w_out[...], preferred_element_type=jnp.float32) + b_out[...]
    return x3 + o2.reshape(L, bb, d)


def _fuse_body(pa_ref, pb_ref, *rest):
    (bias2a, bias2b, lin1a_w, lin1a_b, lin1b_w, lin1b_b,
     norm_a_g, norm_a_b, norm_b_g, norm_b_b, fus_w, fus_b) = rest[:12]
    pma = rest[12:23]
    pspec = rest[23:34]
    pmb = rest[34:45]
    out_ref = rest[45]
    scr = rest[46:51]

    def fuse(x3, lw, lb, ng, nb, mparams):
        L, bb, d = x3.shape
        xp2 = jnp.dot(x3.reshape(L * bb, d), lw[...],
                      preferred_element_type=jnp.float32) + lb[...]
        m3 = xp2.reshape(L, bb, d)
        for mp in mparams:
            m3 = _mamba(m3, mp, scr)
        x3 = x3 + m3
        h = jax.nn.relu(_lnk(x3.reshape(L * bb, d), ng[...], nb[...]))
        return h

    xa3 = pa_ref[...] + bias2a[...][:, None, :]
    xb3 = pb_ref[...] + bias2b[...][:, None, :]
    fa = fuse(xa3, lin1a_w, lin1a_b, norm_a_g, norm_a_b, (pma, pspec))
    fb = fuse(xb3, lin1b_w, lin1b_b, norm_b_g, norm_b_b, (pmb,))
    cat = jnp.concatenate([fa, fb], axis=-1)
    o = jnp.dot(cat, fus_w[...], preferred_element_type=jnp.float32) + fus_b[...]
    out_ref[...] = jnp.swapaxes(o.reshape(_L, _BB, _D), 0, 1)


def _wspec(a):
    nd = a.ndim
    return pl.BlockSpec(a.shape, lambda i, _nd=nd: (0,) * _nd)


def _fold_bn(w, b, g, beta, m, v):
    s = g * jax.lax.rsqrt(v + _EPS)
    return w * s[:, None, None, None], (b - m) * s + beta


def _prep_mamba(p):
    dt_rank = p['w_dt'].shape[0]
    wbig = jnp.concatenate(
        [p['w_x'][:, :dt_rank] @ p['w_dt'],
         p['w_x'][:, dt_rank:dt_rank + _N],
         p['w_x'][:, dt_rank + _N:dt_rank + 2 * _N]], axis=1)
    at = -jnp.exp(p['a_log']).T
    r = lambda a: a.reshape(1, -1)
    return [r(p['ln_g']), r(p['ln_b']), p['w_in'], p['conv_w'].T,
            r(p['conv_b']), wbig, r(p['b_dt']), at, r(p['dskip']),
            p['w_out'], r(p['b_out'])]


def kernel(x1, x2, cnn, emb, pos_embed, fm1, fm2, fusion):
    bf16 = jnp.bfloat16
    w1a, b1a = _fold_bn(cnn['c1_w'], cnn['c1_b'], cnn['c1_g'],
                        cnn['c1_beta'], cnn['c1_m'], cnn['c1_v'])
    w1b, b1b = _fold_bn(cnn['c2_w'], cnn['c2_b'], cnn['c2_g'],
                        cnn['c2_beta'], cnn['c2_m'], cnn['c2_v'])
    w2a, b2a = _fold_bn(cnn['c1b_w'], cnn['c1b_b'], cnn['c1b_g'],
                        cnn['c1b_beta'], cnn['c1b_m'], cnn['c1b_v'])
    w2b, b2b = _fold_bn(cnn['c2b_w'], cnn['c2b_b'], cnn['c2b_g'],
                        cnn['c2b_beta'], cnn['c2b_m'], cnn['c2b_v'])
    w1c = jnp.zeros((3, 3, 31, 64), jnp.float32)
    w1c = w1c.at[:, :, :30, :32].set(w1a.transpose(2, 3, 1, 0))
    w1c = w1c.at[:, :, 30:, 32:].set(w1b.transpose(2, 3, 1, 0))
    w1c = w1c.reshape(9 * 31, 64).astype(bf16)
    b1c = jnp.concatenate([b1a, b1b]).reshape(1, 64)
    w2c = jnp.zeros((3, 3, 64, 128), jnp.float32)
    w2c = w2c.at[:, :, :32, :64].set(w2a.transpose(2, 3, 1, 0))
    w2c = w2c.at[:, :, 32:, 64:].set(w2b.transpose(2, 3, 1, 0))
    w2c = w2c.reshape(576, 128).astype(bf16)
    b2c = jnp.concatenate([b2a, b2b]).reshape(1, 128)

    xcat = jnp.concatenate(
        [jnp.transpose(x1, (2, 3, 0, 1)),
         jnp.transpose(x2, (2, 3, 0, 1))], axis=-1).astype(bf16)

    kp = jnp.asarray(_KP)
    m1ta = (emb['w1'].T @ kp).astype(bf16)
    m1tb = (emb['w2'].T @ kp).astype(bf16)
    bias2a = emb['b1'][:, None] + pos_embed[0, :_L, :]
    bias2b = emb['b2'][:, None] + pos_embed[0, :_L, :]

    pma = _prep_mamba(fm1['ms'])
    pspec = _prep_mamba(fm1['spec'])
    pmb = _prep_mamba(fm2['ms'])
    r = lambda a: a.reshape(1, -1)
    weights = ([bias2a, bias2b, fm1['lin1_w'], r(fm1['lin1_b']),
                fm2['lin1_w'], r(fm2['lin1_b']),
                r(fm1['norm_g']), r(fm1['norm_b']),
                r(fm2['norm_g']), r(fm2['norm_b']),
                fusion['w'], r(fusion['b'])] + pma + pspec + pmb)

    nblk = _B // _BB
    cparams = pltpu.CompilerParams(dimension_semantics=("parallel",),
                                   vmem_limit_bytes=56 * 2 ** 20)

    a2, b2_ = pl.pallas_call(
        _conv_body,
        grid=(nblk,),
        in_specs=[pl.BlockSpec((13, 13, _BB, 31), lambda i: (0, 0, i, 0)),
                  _wspec(w1c), _wspec(b1c), _wspec(w2c), _wspec(b2c)],
        out_specs=[pl.BlockSpec((_HW, _BB, 64), lambda i: (0, i, 0))] * 2,
        out_shape=[jax.ShapeDtypeStruct((_HW, _B, 64), bf16)] * 2,
        compiler_params=cparams,
    )(xcat, w1c, b1c, w2c, b2c)

    xa = a2.reshape(_HW, _B * _D)
    xb = b2_.reshape(_HW, _B * _D)
    pa, pb = pl.pallas_call(
        _pool_body,
        grid=(_B * _D // _CN,),
        in_specs=[pl.BlockSpec((_HW, _CN), lambda i: (0, i)),
                  pl.BlockSpec((_HW, _CN), lambda i: (0, i)),
                  _wspec(m1ta), _wspec(m1tb)],
        out_specs=[pl.BlockSpec((_L, _CN), lambda i: (0, i))] * 2,
        out_shape=[jax.ShapeDtypeStruct((_L, _B * _D), jnp.float32)] * 2,
        compiler_params=cparams,
    )(xa, xb, m1ta, m1tb)

    pa3 = pa.reshape(_L, _B, _D)
    pb3 = pb.reshape(_L, _B, _D)
    out = pl.pallas_call(
        _fuse_body,
        grid=(nblk,),
        in_specs=([pl.BlockSpec((_L, _BB, _D), lambda i: (0, i, 0))] * 2 +
                  [_wspec(w) for w in weights]),
        out_specs=pl.BlockSpec((_BB, _L, _D), lambda i: (i, 0, 0)),
        out_shape=jax.ShapeDtypeStruct((_B, _L, _D), jnp.float32),
        scratch_shapes=[pltpu.VMEM((_L, _BB, _N, _D), jnp.float32)] * 2 +
                       [pltpu.VMEM((_L, _BB, _D), jnp.float32),
                        pltpu.VMEM((_L, _BB, _N), jnp.float32),
                        pltpu.VMEM((_L + 3, _BB, _D), jnp.float32)],
        compiler_params=cparams,
    )(pa3, pb3, *weights)

    return out
